# Initial kernel scaffold; baseline (speedup 1.0000x reference)
import jax
import jax.numpy as jnp
from jax.experimental import pallas as pl


def kernel(topo_x, topo_edge_index, topo_batch, lc_x, lc_edge_index, lc_batch, topo_W0, topo_b0, topo_W1, topo_b1, topo_W2, topo_b2, lc_W0, lc_b0, lc_W1, lc_b1, lc_W2, lc_b2, ml_W0, ml_b0, ml_W1, ml_b1, ml_W2, ml_b2):
    raise NotImplementedError("write your pallas kernel here")


# speedup vs baseline: 4.1618x; 4.1618x over previous
import functools

import jax
import jax.numpy as jnp
from jax import lax
from jax.experimental import pallas as pl
from jax.experimental.pallas import tpu as pltpu
from jax.experimental.pallas import tpu_sc as plsc

N = 10000
N_PAD = 10240
K = 500
NEG_INF = -3.0e38


def _sc_deg(e_pad, n_pad):
    nw = 32
    per_tile = e_pad // nw
    n_chunks = per_tile // 128
    rows_sub = n_pad // 16
    mesh = plsc.VectorSubcoreMesh(core_axis_name="c", subcore_axis_name="s")

    @functools.partial(
        pl.kernel,
        mesh=mesh,
        out_type=jax.ShapeDtypeStruct((2, n_pad, 128), jnp.float32),
        scratch_types=[
            pltpu.VMEM((128,), jnp.int32),
            pltpu.VMEM((128, 128), jnp.float32),
            pltpu.VMEM_SHARED((n_pad, 128), jnp.float32),
        ],
    )
    def k(dst_hbm, zeros_hbm, ones_hbm, out_hbm, dst_v, ones_v, acc_sh):
        cid = lax.axis_index("c")
        sid = lax.axis_index("s")
        wid = cid * 16 + sid
        r0 = sid * rows_sub
        pltpu.sync_copy(zeros_hbm.at[pl.ds(r0, rows_sub)],
                        acc_sh.at[pl.ds(r0, rows_sub)])
        pltpu.sync_copy(ones_hbm, ones_v)
        plsc.subcore_barrier()

        def body(i, c):
            off = wid * per_tile + i * 128
            pltpu.sync_copy(dst_hbm.at[pl.ds(off, 128)], dst_v)
            pltpu.sync_copy(ones_v, acc_sh.at[dst_v], add=True)
            return c

        lax.fori_loop(0, n_chunks, body, 0)
        plsc.subcore_barrier()

        @pl.when(cid == 0)
        def _():
            pltpu.sync_copy(acc_sh.at[pl.ds(r0, rows_sub)],
                            out_hbm.at[0, pl.ds(r0, rows_sub)])

        @pl.when(cid == 1)
        def _():
            pltpu.sync_copy(acc_sh.at[pl.ds(r0, rows_sub)],
                            out_hbm.at[1, pl.ds(r0, rows_sub)])

    return k


def _sc_msg(e_pad, n_pad, ch):
    nw = 32
    per_tile = e_pad // nw
    n_chunks = per_tile // 128
    rows_sub = n_pad // 16
    mesh = plsc.VectorSubcoreMesh(core_axis_name="c", subcore_axis_name="s")

    @functools.partial(
        pl.kernel,
        mesh=mesh,
        out_type=jax.ShapeDtypeStruct((2, n_pad, ch), jnp.float32),
        scratch_types=[
            pltpu.VMEM((128,), jnp.int32),
            pltpu.VMEM((128,), jnp.int32),
            pltpu.VMEM((128, ch), jnp.float32),
            pltpu.VMEM_SHARED((n_pad, ch), jnp.float32),
            pltpu.SemaphoreType.DMA,
        ],
    )
    def k(src_hbm, dst_hbm, h_hbm, zeros_hbm, out_hbm,
          src_v, dst_v, rows_v, acc_sh, sem):
        cid = lax.axis_index("c")
        sid = lax.axis_index("s")
        wid = cid * 16 + sid
        r0 = sid * rows_sub

        @pl.when(cid == 0)
        def _():
            pltpu.sync_copy(h_hbm.at[pl.ds(r0, rows_sub)],
                            acc_sh.at[pl.ds(r0, rows_sub)])

        @pl.when(cid == 1)
        def _():
            pltpu.sync_copy(zeros_hbm.at[pl.ds(r0, rows_sub)],
                            acc_sh.at[pl.ds(r0, rows_sub)])

        plsc.subcore_barrier()

        def body(i, c):
            off = wid * per_tile + i * 128
            pltpu.sync_copy(src_hbm.at[pl.ds(off, 128)], src_v)
            pltpu.sync_copy(dst_hbm.at[pl.ds(off, 128)], dst_v)
            pltpu.async_copy(h_hbm.at[src_v], rows_v, sem).wait()
            pltpu.sync_copy(rows_v, acc_sh.at[dst_v], add=True)
            return c

        lax.fori_loop(0, n_chunks, body, 0)
        plsc.subcore_barrier()

        @pl.when(cid == 0)
        def _():
            pltpu.sync_copy(acc_sh.at[pl.ds(r0, rows_sub)],
                            out_hbm.at[0, pl.ds(r0, rows_sub)])

        @pl.when(cid == 1)
        def _():
            pltpu.sync_copy(acc_sh.at[pl.ds(r0, rows_sub)],
                            out_hbm.at[1, pl.ds(r0, rows_sub)])

    return k


def _dinv_prep(d0, d1):
    n = d0.shape[0]

    def body(a_ref, b_ref, o_ref):
        o_ref[...] = jnp.broadcast_to(
            lax.rsqrt(a_ref[:, 0:1] + b_ref[:, 0:1] + 1.0), o_ref.shape)

    return pl.pallas_call(
        body,
        grid=(n // 1024,),
        in_specs=[pl.BlockSpec((1024, 128), lambda i: (i, 0)),
                  pl.BlockSpec((1024, 128), lambda i: (i, 0))],
        out_specs=pl.BlockSpec((1024, 16), lambda i: (i, 0)),
        out_shape=jax.ShapeDtypeStruct((n, 16), jnp.float32),
    )(d0, d1)


def _mm_first(x, w, dinv2d):
    n, din = x.shape
    dout = w.shape[1]

    def body(x_ref, w_ref, dv_ref, o_ref):
        d = dv_ref[:, 0:1]
        o_ref[...] = d * jnp.dot(x_ref[...], w_ref[...],
                                 preferred_element_type=jnp.float32)

    return pl.pallas_call(
        body,
        grid=(n // 1024,),
        in_specs=[pl.BlockSpec((1024, din), lambda i: (i, 0)),
                  pl.BlockSpec((din, dout), lambda i: (0, 0)),
                  pl.BlockSpec((1024, 16), lambda i: (i, 0))],
        out_specs=pl.BlockSpec((1024, dout), lambda i: (i, 0)),
        out_shape=jax.ShapeDtypeStruct((n, dout), jnp.float32),
    )(x, w, dinv2d)


def _mm_mid(p0, p1, b2d, w, dinv2d):
    n, din = p0.shape
    dout = w.shape[1]

    def body(p0_ref, p1_ref, b_ref, w_ref, dv_ref, o_ref):
        d = dv_ref[:, 0:1]
        u = jnp.maximum(d * (p0_ref[...] + p1_ref[...]) + b_ref[...], 0.0)
        o_ref[...] = d * jnp.dot(u, w_ref[...],
                                 preferred_element_type=jnp.float32)

    return pl.pallas_call(
        body,
        grid=(n // 1024,),
        in_specs=[pl.BlockSpec((1024, din), lambda i: (i, 0)),
                  pl.BlockSpec((1024, din), lambda i: (i, 0)),
                  pl.BlockSpec((1, din), lambda i: (0, 0)),
                  pl.BlockSpec((din, dout), lambda i: (0, 0)),
                  pl.BlockSpec((1024, 16), lambda i: (i, 0))],
        out_specs=pl.BlockSpec((1024, dout), lambda i: (i, 0)),
        out_shape=jax.ShapeDtypeStruct((n, dout), jnp.float32),
    )(p0, p1, b2d, w, dinv2d)


def _sortpool(r0, r1, dinv2d, b2d, real_ch=64):
    n, ch = r0.shape
    nchunk = n // 128

    def body(r0_ref, r1_ref, dv_ref, b_ref, o_ref, z_scr, keys_scr, ranks_scr):
        ident = (lax.broadcasted_iota(jnp.int32, (128, 128), 0)
                 == lax.broadcasted_iota(jnp.int32, (128, 128), 1)
                 ).astype(jnp.float32)
        e_last = (lax.broadcasted_iota(jnp.int32, (1, ch), 1)
                  == (real_ch - 1)).astype(jnp.float32)
        b = b_ref[...]

        def phase_a(it, c):
            sl = pl.ds(it * 128, 128)
            d = dv_ref[sl, 0:1]
            zt = d * (r0_ref[sl, :] + r1_ref[sl, :]) + b
            z_scr[sl, :] = zt
            kr = lax.dot_general(e_last, zt, (((1,), (1,)), ((), ())),
                                 preferred_element_type=jnp.float32, precision=lax.Precision.HIGHEST)
            glob = it * 128 + lax.broadcasted_iota(jnp.int32, (1, 128), 1)
            kr = jnp.where(glob < N, kr, NEG_INF)
            keys_scr[0:1, pl.ds(it * 128, 128)] = kr
            return c

        lax.fori_loop(0, nchunk, phase_a, 0)

        def phase_b(ic, c):
            krow_i = keys_scr[0:1, pl.ds(ic * 128, 128)]
            ki = lax.dot_general(ident, krow_i, (((1,), (1,)), ((), ())),
                                 preferred_element_type=jnp.float32, precision=lax.Precision.HIGHEST)
            iglob = (ic * 128
                     + lax.broadcasted_iota(jnp.int32, (128, 1), 0))

            def inner(jt, cnt):
                kj = keys_scr[0:1, pl.ds(jt * 128, 128)]
                jglob = (jt * 128
                         + lax.broadcasted_iota(jnp.int32, (1, 128), 1))
                gt = kj > ki
                eq = (kj == ki) & (jglob < iglob)
                contrib = jnp.where(gt | eq, 1.0, 0.0)
                return cnt + jnp.sum(contrib, axis=1, keepdims=True)

            cnt = lax.fori_loop(0, nchunk, inner,
                                jnp.zeros((128, 1), jnp.float32))
            rrow = lax.dot_general(cnt, ident, (((0,), (0,)), ((), ())),
                                   preferred_element_type=jnp.float32, precision=lax.Precision.HIGHEST)
            ranks_scr[0:1, pl.ds(ic * 128, 128)] = rrow
            return c

        lax.fori_loop(0, nchunk, phase_b, 0)

        o_ref[...] = jnp.zeros((512, real_ch), jnp.float32)

        def phase_c(jt, c):
            rr = ranks_scr[0:1, pl.ds(jt * 128, 128)]
            zc = z_scr[pl.ds(jt * 128, 128), 0:real_ch]
            for rt in range(4):
                pcol = (rt * 128
                        + lax.broadcasted_iota(jnp.int32, (128, 1), 0)
                        ).astype(jnp.float32)
                sel = jnp.where(rr == pcol, 1.0, 0.0)
                o_ref[rt * 128:(rt + 1) * 128, :] += jnp.dot(
                    sel, zc, preferred_element_type=jnp.float32, precision=lax.Precision.HIGHEST)
            return c

        lax.fori_loop(0, nchunk, phase_c, 0)

    return pl.pallas_call(
        body,
        out_shape=jax.ShapeDtypeStruct((512, real_ch), jnp.float32),
        scratch_shapes=[
            pltpu.VMEM((n, ch), jnp.float32),
            pltpu.VMEM((1, n), jnp.float32),
            pltpu.VMEM((1, n), jnp.float32),
        ],
    )(r0, r1, dinv2d, b2d)


def _mlp(v3d, w0, b0, w1, b1, w2p, b2p):
    nblk, _, kblk = v3d.shape

    def body(v_ref, w0_ref, b0_ref, w1_ref, b1_ref, w2_ref, b2_ref,
             o_ref, acc):
        i = pl.program_id(0)

        @pl.when(i == 0)
        def _():
            acc[...] = jnp.zeros_like(acc)

        v = v_ref[...].reshape(1, kblk)
        acc[...] += jnp.dot(v, w0_ref[...], preferred_element_type=jnp.float32)

        @pl.when(i == nblk - 1)
        def _():
            h0 = jnp.maximum(acc[...] + b0_ref[...], 0.0)
            h1 = jnp.maximum(
                jnp.dot(h0, w1_ref[...], preferred_element_type=jnp.float32)
                + b1_ref[...], 0.0)
            o_ref[...] = jnp.dot(
                h1, w2_ref[...], preferred_element_type=jnp.float32
            ) + b2_ref[...]

    return pl.pallas_call(
        body,
        grid=(nblk,),
        in_specs=[pl.BlockSpec((1, 1, kblk), lambda i: (i, 0, 0)),
                  pl.BlockSpec((kblk, 256), lambda i: (i, 0)),
                  pl.BlockSpec((1, 256), lambda i: (0, 0)),
                  pl.BlockSpec((256, 128), lambda i: (0, 0)),
                  pl.BlockSpec((1, 128), lambda i: (0, 0)),
                  pl.BlockSpec((128, 128), lambda i: (0, 0)),
                  pl.BlockSpec((1, 128), lambda i: (0, 0))],
        out_specs=pl.BlockSpec((1, 128), lambda i: (0, 0)),
        out_shape=jax.ShapeDtypeStruct((1, 128), jnp.float32),
        scratch_shapes=[pltpu.VMEM((1, 256), jnp.float32)],
    )(v3d, w0, b0, w1, b1, w2p, b2p)


def _run_gnn(x, ei, w0, b0, w1, b1, w2, b2):
    e = ei.shape[1]
    e_pad = ((e + 4095) // 4096) * 4096
    pad = e_pad - e
    src = jnp.concatenate([ei[0], jnp.full((pad,), N, jnp.int32)])
    dst = jnp.concatenate([ei[1], jnp.full((pad,), N, jnp.int32)])
    x_p = jnp.pad(x, ((0, N_PAD - N), (0, 0)))

    zeros128 = jnp.zeros((N_PAD, 128), jnp.float32)
    ones128 = jnp.ones((128, 128), jnp.float32)

    degp = _sc_deg(e_pad, N_PAD)(dst, zeros128, ones128)
    dinv2d = _dinv_prep(degp[0], degp[1])

    h0 = _mm_first(x_p, w0, dinv2d)
    p = _sc_msg(e_pad, N_PAD, 128)(src, dst, h0, zeros128)
    h1 = _mm_mid(p[0], p[1], b0.reshape(1, -1), w1, dinv2d)
    q = _sc_msg(e_pad, N_PAD, 128)(src, dst, h1, zeros128)
    w2p = jnp.pad(w2, ((0, 0), (0, 128 - w2.shape[1])))
    h2 = _mm_mid(q[0], q[1], b1.reshape(1, -1), w2p, dinv2d)
    r = _sc_msg(e_pad, N_PAD, 128)(src, dst, h2, zeros128)
    b2p = jnp.pad(b2, (0, 128 - b2.shape[0])).reshape(1, -1)
    pooled = _sortpool(r[0], r[1], dinv2d, b2p, real_ch=b2.shape[0])
    return pooled[:K].reshape(-1)


def kernel(topo_x, topo_edge_index, topo_batch, lc_x, lc_edge_index, lc_batch,
           topo_W0, topo_b0, topo_W1, topo_b1, topo_W2, topo_b2,
           lc_W0, lc_b0, lc_W1, lc_b1, lc_W2, lc_b2,
           ml_W0, ml_b0, ml_W1, ml_b1, ml_W2, ml_b2):
    t = _run_gnn(topo_x, topo_edge_index,
                 topo_W0, topo_b0, topo_W1, topo_b1, topo_W2, topo_b2)
    l = _run_gnn(lc_x, lc_edge_index,
                 lc_W0, lc_b0, lc_W1, lc_b1, lc_W2, lc_b2)
    v = jnp.concatenate([t, l]).reshape(32, 1, 2000)
    w2p = jnp.pad(ml_W2, ((0, 0), (0, 127)))
    b2p = jnp.pad(ml_b2, (0, 127)).reshape(1, 128)
    out = _mlp(v, ml_W0, ml_b0.reshape(1, 256), ml_W1,
               ml_b1.reshape(1, 128), w2p, b2p)
    return out[0, :1]
